# baseline (speedup 1.0000x reference)
import functools

import jax
import jax.numpy as jnp
from jax import lax
from jax.experimental import pallas as pl
from jax.experimental.pallas import tpu as pltpu
from jax.experimental.pallas import tpu_sc as plsc

_NUM_CLASSES = 1000000
_HID = 32
_BATCH = 16384
_NC = 2
_NS = 16
_NW = _NC * _NS
_CW = 512
_RANGE = 31232
_TAIL_LO = 999936
_L = 16

_mesh = plsc.VectorSubcoreMesh(core_axis_name="c", subcore_axis_name="s")


def _iota():
    return lax.iota(jnp.int32, _L)


def _full(v):
    return jnp.full((_L,), v, jnp.int32)


@functools.partial(
    pl.kernel,
    mesh=_mesh,
    out_type=jax.ShapeDtypeStruct((_BATCH, 128), jnp.float32),
    scratch_types=[
        pltpu.VMEM((_BATCH,), jnp.int32),
        pltpu.VMEM((_BATCH,), jnp.int32),
        pltpu.VMEM((_BATCH + 256,), jnp.int32),
        pltpu.VMEM((_BATCH + 256,), jnp.int32),
        pltpu.VMEM((128,), jnp.int32),
        pltpu.VMEM((128,), jnp.int32),
        pltpu.VMEM((128,), jnp.int32),
        pltpu.VMEM((_HID, _CW), jnp.float32),
        pltpu.VMEM((_HID, _CW), jnp.float32),
        pltpu.VMEM((72, 128), jnp.float32),
        pltpu.VMEM((128, 128), jnp.float32),
        pltpu.SemaphoreType.DMA,
        pltpu.SemaphoreType.DMA,
        pltpu.SemaphoreType.DMA,
    ],
    compiler_params=pltpu.CompilerParams(needs_layout_passes=False),
)
def _embed(lab_hbm, drop_hbm, table_t, tail_hbm, out_hbm,
           lab_v, drop_v, slab_v, spos_v, cnts_v, offs_v, posb_v,
           buf0, buf1, tail_v, extbuf, sem0, sem1, semf):
    wid = lax.axis_index("s") * _NC + lax.axis_index("c")
    is31 = wid == _NW - 1
    lo = wid * _RANGE
    hi = jnp.where(is31, _NUM_CLASSES + 1, lo + _RANGE)
    nch = jnp.where(is31, 62, 61)
    ones = jnp.ones((_L,), jnp.int32)
    negones = _full(-1)

    def issue(c, buf, sem):
        col = pl.multiple_of(lo + c * _CW, _CW)
        return pltpu.async_copy(table_t.at[:, pl.ds(col, _CW)], buf, sem)

    issue(0, buf0, sem0)
    issue(1, buf1, sem1)

    pltpu.sync_copy(lab_hbm, lab_v)
    pltpu.sync_copy(drop_hbm, drop_v)
    pltpu.sync_copy(tail_hbm, tail_v)

    for k in range(8):
        cnts_v[pl.ds(k * _L, _L)] = jnp.zeros((_L,), jnp.int32)

    def in_range_q(idx):
        inr = (idx >= lo) & (idx < hi)
        q = jnp.minimum(lax.shift_right_logical(idx - lo, 9), 62)
        q = jnp.where(inr, q, 0)
        return inr, q

    def hist_body(k4, _):
        for u in range(4):
            k = k4 * 4 + u
            sl = pl.ds(k * _L, _L)
            idx = jnp.where(drop_v[sl] != 0, _NUM_CLASSES, lab_v[sl])
            lab_v[sl] = idx
            inr, q = in_range_q(idx)
            plsc.addupdate_scatter(cnts_v, [q], ones, mask=inr)
        return 0
    lax.fori_loop(0, _BATCH // _L // 4, hist_body, 0)

    carry = jnp.int32(0)
    for k in range(8):
        sl = pl.ds(k * _L, _L)
        c = cnts_v[sl]
        offs_v[sl] = plsc.cumsum(c) - c + carry
        carry = carry + jnp.sum(c)

    base_rank = jnp.min(plsc.scan_count(jnp.zeros((_L,), jnp.int32))[0])

    def place_body(k4, _):
        for u in range(4):
            k = k4 * 4 + u
            idx = lab_v[pl.ds(k * _L, _L)]
            inr, q = in_range_q(idx)
            gpos = k * _L + _iota()
            base = plsc.load_gather(offs_v, [q])
            rank = plsc.scan_count(q, mask=inr)[0] - base_rank
            slot = base + rank
            plsc.store_scatter(slab_v, [slot], idx, mask=inr)
            plsc.store_scatter(spos_v, [slot], gpos, mask=inr)
            plsc.addupdate_scatter(offs_v, [q], ones, mask=inr)
        return 0
    lax.fori_loop(0, _BATCH // _L // 4, place_body, 0)

    for k in range(8):
        posb_v[pl.ds(k * _L, _L)] = negones

    def flush():
        pltpu.async_copy(
            extbuf, out_hbm.at[plsc.Indices(posb_v, ignored_value=-1)], semf
        ).wait()
        for k in range(8):
            posb_v[pl.ds(k * _L, _L)] = negones

    def cnt_at(c):
        acc = jnp.int32(0)
        for k in range(4):
            acc = acc + jnp.sum(jnp.where(k * _L + _iota() == c,
                                          cnts_v[pl.ds(k * _L, _L)], 0))
        return acc

    def do_segment(seg_lo, seg_hi, j, src, base_col, transposed):
        a0 = (seg_lo // 8) * 8

        def chunk_body(m, j):
            base_i = a0 + _L * m
            lpos = base_i + _iota()
            msk = (lpos >= seg_lo) & (lpos < seg_hi)
            sl = pl.ds(pl.multiple_of(base_i, 8), _L)
            labs = slab_v[sl]
            poss = spos_v[sl]
            r = labs - base_col
            slotv = j + _iota()
            for h in range(_HID):
                hv = _full(h)
                if transposed:
                    g = plsc.load_gather(src, [hv, r], mask=msk)
                else:
                    g = plsc.load_gather(src, [r, hv], mask=msk)
                plsc.store_scatter(extbuf, [slotv, hv], g, mask=msk)
            plsc.store_scatter(posb_v, [slotv], poss, mask=msk)
            j = j + _L
            pl.when(j == 128)(flush)
            return jnp.where(j == 128, 0, j)

        nchk = jnp.where(seg_hi > seg_lo, (seg_hi - a0 + _L - 1) // _L, 0)
        return lax.fori_loop(0, nchk, chunk_body, j)

    def pair_body(t, carry):
        j, s_lo = carry
        c0 = 2 * t
        pltpu.make_async_copy(table_t.at[:, pl.ds(0, _CW)], buf0, sem0).wait()
        s_hi = s_lo + cnt_at(c0)
        j = do_segment(s_lo, s_hi, j, buf0, lo + c0 * _CW, True)
        s_lo = s_hi

        @pl.when(c0 + 2 < nch)
        def _():
            issue(c0 + 2, buf0, sem0)

        has_odd = c0 + 1 < nch

        @pl.when(has_odd)
        def _():
            pltpu.make_async_copy(
                table_t.at[:, pl.ds(0, _CW)], buf1, sem1).wait()

        s_hi = s_lo + jnp.where(has_odd, cnt_at(c0 + 1), 0)
        j = do_segment(s_lo, s_hi, j, buf1, lo + (c0 + 1) * _CW, True)
        s_lo = s_hi

        @pl.when(c0 + 3 < nch)
        def _():
            issue(c0 + 3, buf1, sem1)

        return (j, s_lo)

    j, s_lo = lax.fori_loop(0, (nch + 1) // 2, pair_body,
                            (jnp.int32(0), jnp.int32(0)))

    j = do_segment(s_lo, s_lo + cnt_at(62), j, tail_v, _TAIL_LO, False)

    pl.when(j > 0)(flush)


def kernel(labels, train, force_drop_ids, table):
    lab = labels.astype(jnp.int32)
    drop = force_drop_ids.astype(jnp.int32)
    table_t = jnp.swapaxes(table, 0, 1)
    tail = lax.pad(
        lax.slice(table, (_TAIL_LO, 0), (_NUM_CLASSES + 1, _HID)),
        jnp.float32(0.0), ((0, 7, 0), (0, 96, 0)))
    out_wide = _embed(lab, drop, table_t, tail)
    return lax.slice(out_wide, (0, 0), (_BATCH, _HID))


# speedup vs baseline: 1.0029x; 1.0029x over previous
import functools

import jax
import jax.numpy as jnp
from jax import lax
from jax.experimental import pallas as pl
from jax.experimental.pallas import tpu as pltpu
from jax.experimental.pallas import tpu_sc as plsc

_NUM_CLASSES = 1000000
_HID = 32
_BATCH = 16384
_NC = 2
_NS = 16
_NW = _NC * _NS
_CW = 512
_RANGE = 31232
_TAIL_LO = 999936
_L = 16

_mesh = plsc.VectorSubcoreMesh(core_axis_name="c", subcore_axis_name="s")


def _iota():
    return lax.iota(jnp.int32, _L)


def _full(v):
    return jnp.full((_L,), v, jnp.int32)


@functools.partial(
    pl.kernel,
    mesh=_mesh,
    out_type=jax.ShapeDtypeStruct((_BATCH, 128), jnp.float32),
    scratch_types=[
        pltpu.VMEM((_BATCH,), jnp.int32),
        pltpu.VMEM((_BATCH,), jnp.int32),
        pltpu.VMEM((_BATCH + 256,), jnp.int32),
        pltpu.VMEM((_BATCH + 256,), jnp.int32),
        pltpu.VMEM((128,), jnp.int32),
        pltpu.VMEM((128,), jnp.int32),
        pltpu.VMEM((128,), jnp.int32),
        pltpu.VMEM((_HID, _CW), jnp.float32),
        pltpu.VMEM((_HID, _CW), jnp.float32),
        pltpu.VMEM((72, 128), jnp.float32),
        pltpu.VMEM((128, 128), jnp.float32),
        pltpu.SemaphoreType.DMA,
        pltpu.SemaphoreType.DMA,
        pltpu.SemaphoreType.DMA,
    ],
    compiler_params=pltpu.CompilerParams(needs_layout_passes=False),
)
def _embed(lab_hbm, drop_hbm, table_t, tail_hbm, out_hbm,
           lab_v, drop_v, slab_v, spos_v, cnts_v, offs_v, posb_v,
           buf0, buf1, tail_v, extbuf, sem0, sem1, semf):
    wid = lax.axis_index("s") * _NC + lax.axis_index("c")
    is31 = wid == _NW - 1
    lo = wid * _RANGE
    hi = jnp.where(is31, _NUM_CLASSES + 1, lo + _RANGE)
    nch = jnp.where(is31, 62, 61)
    ones = jnp.ones((_L,), jnp.int32)
    negones = _full(-1)

    def issue(c, buf, sem):
        col = pl.multiple_of(lo + c * _CW, _CW)
        return pltpu.async_copy(table_t.at[:, pl.ds(col, _CW)], buf, sem)

    issue(0, buf0, sem0)
    issue(1, buf1, sem1)

    pltpu.sync_copy(lab_hbm, lab_v)
    pltpu.sync_copy(drop_hbm, drop_v)
    pltpu.sync_copy(tail_hbm, tail_v)

    for k in range(8):
        cnts_v[pl.ds(k * _L, _L)] = jnp.zeros((_L,), jnp.int32)

    def in_range_q(idx):
        inr = (idx >= lo) & (idx < hi)
        q = jnp.minimum(lax.shift_right_logical(idx - lo, 9), 62)
        q = jnp.where(inr, q, 0)
        return inr, q

    def hist_body(k4, _):
        for u in range(8):
            k = k4 * 8 + u
            sl = pl.ds(k * _L, _L)
            idx = jnp.where(drop_v[sl] != 0, _NUM_CLASSES, lab_v[sl])
            lab_v[sl] = idx
            inr, q = in_range_q(idx)
            plsc.addupdate_scatter(cnts_v, [q], ones, mask=inr)
        return 0
    lax.fori_loop(0, _BATCH // _L // 8, hist_body, 0)

    carry = jnp.int32(0)
    for k in range(8):
        sl = pl.ds(k * _L, _L)
        c = cnts_v[sl]
        offs_v[sl] = plsc.cumsum(c) - c + carry
        carry = carry + jnp.sum(c)

    base_rank = jnp.min(plsc.scan_count(jnp.zeros((_L,), jnp.int32))[0])

    def place_body(k4, _):
        for u in range(8):
            k = k4 * 8 + u
            idx = lab_v[pl.ds(k * _L, _L)]
            inr, q = in_range_q(idx)
            gpos = k * _L + _iota()
            base = plsc.load_gather(offs_v, [q])
            rank = plsc.scan_count(q, mask=inr)[0] - base_rank
            slot = base + rank
            plsc.store_scatter(slab_v, [slot], idx, mask=inr)
            plsc.store_scatter(spos_v, [slot], gpos, mask=inr)
            plsc.addupdate_scatter(offs_v, [q], ones, mask=inr)
        return 0
    lax.fori_loop(0, _BATCH // _L // 8, place_body, 0)

    for k in range(8):
        posb_v[pl.ds(k * _L, _L)] = negones

    def flush():
        pltpu.async_copy(
            extbuf, out_hbm.at[plsc.Indices(posb_v, ignored_value=-1)], semf
        ).wait()
        for k in range(8):
            posb_v[pl.ds(k * _L, _L)] = negones

    def cnt_at(c):
        acc = jnp.int32(0)
        for k in range(4):
            acc = acc + jnp.sum(jnp.where(k * _L + _iota() == c,
                                          cnts_v[pl.ds(k * _L, _L)], 0))
        return acc

    def do_segment(seg_lo, seg_hi, j, src, base_col, transposed):
        a0 = (seg_lo // 8) * 8

        def chunk_body(m, j):
            base_i = a0 + _L * m
            lpos = base_i + _iota()
            msk = (lpos >= seg_lo) & (lpos < seg_hi)
            sl = pl.ds(pl.multiple_of(base_i, 8), _L)
            labs = slab_v[sl]
            poss = spos_v[sl]
            r = labs - base_col
            slotv = j + _iota()
            for h in range(_HID):
                hv = _full(h)
                if transposed:
                    g = plsc.load_gather(src, [hv, r], mask=msk)
                else:
                    g = plsc.load_gather(src, [r, hv], mask=msk)
                plsc.store_scatter(extbuf, [slotv, hv], g, mask=msk)
            plsc.store_scatter(posb_v, [slotv], poss, mask=msk)
            j = j + _L
            pl.when(j == 128)(flush)
            return jnp.where(j == 128, 0, j)

        nchk = jnp.where(seg_hi > seg_lo, (seg_hi - a0 + _L - 1) // _L, 0)
        return lax.fori_loop(0, nchk, chunk_body, j)

    def pair_body(t, carry):
        j, s_lo = carry
        c0 = 2 * t
        pltpu.make_async_copy(table_t.at[:, pl.ds(0, _CW)], buf0, sem0).wait()
        s_hi = s_lo + cnt_at(c0)
        j = do_segment(s_lo, s_hi, j, buf0, lo + c0 * _CW, True)
        s_lo = s_hi

        @pl.when(c0 + 2 < nch)
        def _():
            issue(c0 + 2, buf0, sem0)

        has_odd = c0 + 1 < nch

        @pl.when(has_odd)
        def _():
            pltpu.make_async_copy(
                table_t.at[:, pl.ds(0, _CW)], buf1, sem1).wait()

        s_hi = s_lo + jnp.where(has_odd, cnt_at(c0 + 1), 0)
        j = do_segment(s_lo, s_hi, j, buf1, lo + (c0 + 1) * _CW, True)
        s_lo = s_hi

        @pl.when(c0 + 3 < nch)
        def _():
            issue(c0 + 3, buf1, sem1)

        return (j, s_lo)

    j, s_lo = lax.fori_loop(0, (nch + 1) // 2, pair_body,
                            (jnp.int32(0), jnp.int32(0)))

    j = do_segment(s_lo, s_lo + cnt_at(62), j, tail_v, _TAIL_LO, False)

    pl.when(j > 0)(flush)


def kernel(labels, train, force_drop_ids, table):
    lab = labels.astype(jnp.int32)
    drop = force_drop_ids.astype(jnp.int32)
    table_t = jnp.swapaxes(table, 0, 1)
    tail = lax.pad(
        lax.slice(table, (_TAIL_LO, 0), (_NUM_CLASSES + 1, _HID)),
        jnp.float32(0.0), ((0, 7, 0), (0, 96, 0)))
    out_wide = _embed(lab, drop, table_t, tail)
    return lax.slice(out_wide, (0, 0), (_BATCH, _HID))


# speedup vs baseline: 1.0042x; 1.0012x over previous
import functools

import jax
import jax.numpy as jnp
from jax import lax
from jax.experimental import pallas as pl
from jax.experimental.pallas import tpu as pltpu
from jax.experimental.pallas import tpu_sc as plsc

_NUM_CLASSES = 1000000
_HID = 32
_BATCH = 16384
_NC = 2
_NS = 16
_NW = _NC * _NS
_CW = 512
_RANGE = 31232
_TAIL_LO = 999936
_L = 16

_mesh = plsc.VectorSubcoreMesh(core_axis_name="c", subcore_axis_name="s")


def _iota():
    return lax.iota(jnp.int32, _L)


def _full(v):
    return jnp.full((_L,), v, jnp.int32)


@functools.partial(
    pl.kernel,
    mesh=_mesh,
    out_type=jax.ShapeDtypeStruct((_BATCH, 128), jnp.float32),
    scratch_types=[
        pltpu.VMEM((_BATCH,), jnp.int32),
        pltpu.VMEM((_BATCH,), jnp.int32),
        pltpu.VMEM((_BATCH + 256,), jnp.int32),
        pltpu.VMEM((_BATCH + 256,), jnp.int32),
        pltpu.VMEM((128,), jnp.int32),
        pltpu.VMEM((128,), jnp.int32),
        pltpu.VMEM((128,), jnp.int32),
        pltpu.VMEM((128,), jnp.int32),
        pltpu.VMEM((128,), jnp.int32),
        pltpu.VMEM((128,), jnp.int32),
        pltpu.VMEM((_HID, _CW), jnp.float32),
        pltpu.VMEM((_HID, _CW), jnp.float32),
        pltpu.VMEM((72, 128), jnp.float32),
        pltpu.VMEM((128, 128), jnp.float32),
        pltpu.SemaphoreType.DMA,
        pltpu.SemaphoreType.DMA,
        pltpu.SemaphoreType.DMA,
    ],
    compiler_params=pltpu.CompilerParams(needs_layout_passes=False),
)
def _embed(lab_hbm, drop_hbm, table_t, tail_hbm, out_hbm,
           lab_v, drop_v, slab_v, spos_v, cnts_v, cnts_b, cnts_c, cnts_d,
           offs_v, posb_v, buf0, buf1, tail_v, extbuf, sem0, sem1, semf):
    wid = lax.axis_index("s") * _NC + lax.axis_index("c")
    is31 = wid == _NW - 1
    lo = wid * _RANGE
    hi = jnp.where(is31, _NUM_CLASSES + 1, lo + _RANGE)
    nch = jnp.where(is31, 62, 61)
    ones = jnp.ones((_L,), jnp.int32)
    negones = _full(-1)

    def issue(c, buf, sem):
        col = pl.multiple_of(lo + c * _CW, _CW)
        return pltpu.async_copy(table_t.at[:, pl.ds(col, _CW)], buf, sem)

    issue(0, buf0, sem0)
    issue(1, buf1, sem1)

    pltpu.sync_copy(lab_hbm, lab_v)
    pltpu.sync_copy(drop_hbm, drop_v)
    pltpu.sync_copy(tail_hbm, tail_v)

    for k in range(8):
        cnts_v[pl.ds(k * _L, _L)] = jnp.zeros((_L,), jnp.int32)
    for ref in (cnts_b, cnts_c, cnts_d):
        for k in range(4):
            ref[pl.ds(k * _L, _L)] = jnp.zeros((_L,), jnp.int32)

    def in_range_q(idx):
        inr = (idx >= lo) & (idx < hi)
        q = jnp.minimum(lax.shift_right_logical(idx - lo, 9), 62)
        q = jnp.where(inr, q, 0)
        return inr, q

    stripes = (cnts_v, cnts_b, cnts_c, cnts_d)

    def hist_body(k4, _):
        for u in range(8):
            k = k4 * 8 + u
            sl = pl.ds(k * _L, _L)
            idx = jnp.where(drop_v[sl] != 0, _NUM_CLASSES, lab_v[sl])
            lab_v[sl] = idx
            inr, q = in_range_q(idx)
            plsc.addupdate_scatter(stripes[u % 4], [q], ones, mask=inr)
        return 0
    lax.fori_loop(0, _BATCH // _L // 8, hist_body, 0)

    for k in range(4):
        sl = pl.ds(k * _L, _L)
        cnts_v[sl] = cnts_v[sl] + cnts_b[sl] + cnts_c[sl] + cnts_d[sl]

    carry = jnp.int32(0)
    for k in range(8):
        sl = pl.ds(k * _L, _L)
        c = cnts_v[sl]
        offs_v[sl] = plsc.cumsum(c) - c + carry
        carry = carry + jnp.sum(c)

    base_rank = jnp.min(plsc.scan_count(jnp.zeros((_L,), jnp.int32))[0])

    def place_body(k4, _):
        for u in range(8):
            k = k4 * 8 + u
            idx = lab_v[pl.ds(k * _L, _L)]
            inr, q = in_range_q(idx)
            gpos = k * _L + _iota()
            base = plsc.load_gather(offs_v, [q])
            rank = plsc.scan_count(q, mask=inr)[0] - base_rank
            slot = base + rank
            plsc.store_scatter(slab_v, [slot], idx, mask=inr)
            plsc.store_scatter(spos_v, [slot], gpos, mask=inr)
            plsc.addupdate_scatter(offs_v, [q], ones, mask=inr)
        return 0
    lax.fori_loop(0, _BATCH // _L // 8, place_body, 0)

    for k in range(8):
        posb_v[pl.ds(k * _L, _L)] = negones

    def flush():
        pltpu.async_copy(
            extbuf, out_hbm.at[plsc.Indices(posb_v, ignored_value=-1)], semf
        ).wait()
        for k in range(8):
            posb_v[pl.ds(k * _L, _L)] = negones

    def cnt_at(c):
        acc = jnp.int32(0)
        for k in range(4):
            acc = acc + jnp.sum(jnp.where(k * _L + _iota() == c,
                                          cnts_v[pl.ds(k * _L, _L)], 0))
        return acc

    def do_segment(seg_lo, seg_hi, j, src, base_col, transposed):
        a0 = (seg_lo // 8) * 8

        def chunk_body(m, j):
            base_i = a0 + _L * m
            lpos = base_i + _iota()
            msk = (lpos >= seg_lo) & (lpos < seg_hi)
            sl = pl.ds(pl.multiple_of(base_i, 8), _L)
            labs = slab_v[sl]
            poss = spos_v[sl]
            r = labs - base_col
            slotv = j + _iota()
            for h in range(_HID):
                hv = _full(h)
                if transposed:
                    g = plsc.load_gather(src, [hv, r], mask=msk)
                else:
                    g = plsc.load_gather(src, [r, hv], mask=msk)
                plsc.store_scatter(extbuf, [slotv, hv], g, mask=msk)
            plsc.store_scatter(posb_v, [slotv], poss, mask=msk)
            j = j + _L
            pl.when(j == 128)(flush)
            return jnp.where(j == 128, 0, j)

        nchk = jnp.where(seg_hi > seg_lo, (seg_hi - a0 + _L - 1) // _L, 0)
        return lax.fori_loop(0, nchk, chunk_body, j)

    def pair_body(t, carry):
        j, s_lo = carry
        c0 = 2 * t
        pltpu.make_async_copy(table_t.at[:, pl.ds(0, _CW)], buf0, sem0).wait()
        s_hi = s_lo + cnt_at(c0)
        j = do_segment(s_lo, s_hi, j, buf0, lo + c0 * _CW, True)
        s_lo = s_hi

        @pl.when(c0 + 2 < nch)
        def _():
            issue(c0 + 2, buf0, sem0)

        has_odd = c0 + 1 < nch

        @pl.when(has_odd)
        def _():
            pltpu.make_async_copy(
                table_t.at[:, pl.ds(0, _CW)], buf1, sem1).wait()

        s_hi = s_lo + jnp.where(has_odd, cnt_at(c0 + 1), 0)
        j = do_segment(s_lo, s_hi, j, buf1, lo + (c0 + 1) * _CW, True)
        s_lo = s_hi

        @pl.when(c0 + 3 < nch)
        def _():
            issue(c0 + 3, buf1, sem1)

        return (j, s_lo)

    j, s_lo = lax.fori_loop(0, (nch + 1) // 2, pair_body,
                            (jnp.int32(0), jnp.int32(0)))

    j = do_segment(s_lo, s_lo + cnt_at(62), j, tail_v, _TAIL_LO, False)

    pl.when(j > 0)(flush)


def kernel(labels, train, force_drop_ids, table):
    lab = labels.astype(jnp.int32)
    drop = force_drop_ids.astype(jnp.int32)
    table_t = jnp.swapaxes(table, 0, 1)
    tail = lax.pad(
        lax.slice(table, (_TAIL_LO, 0), (_NUM_CLASSES + 1, _HID)),
        jnp.float32(0.0), ((0, 7, 0), (0, 96, 0)))
    out_wide = _embed(lab, drop, table_t, tail)
    return lax.slice(out_wide, (0, 0), (_BATCH, _HID))


# speedup vs baseline: 1.0052x; 1.0011x over previous
import functools

import jax
import jax.numpy as jnp
from jax import lax
from jax.experimental import pallas as pl
from jax.experimental.pallas import tpu as pltpu
from jax.experimental.pallas import tpu_sc as plsc

_NUM_CLASSES = 1000000
_HID = 32
_BATCH = 16384
_NC = 2
_NS = 16
_NW = _NC * _NS
_CW = 512
_RANGE = 31232
_TAIL_LO = 999936
_L = 16

_mesh = plsc.VectorSubcoreMesh(core_axis_name="c", subcore_axis_name="s")


def _iota():
    return lax.iota(jnp.int32, _L)


def _full(v):
    return jnp.full((_L,), v, jnp.int32)


@functools.partial(
    pl.kernel,
    mesh=_mesh,
    out_type=jax.ShapeDtypeStruct((_BATCH, 128), jnp.float32),
    scratch_types=[
        pltpu.VMEM((_BATCH,), jnp.int32),
        pltpu.VMEM((_BATCH,), jnp.int32),
        pltpu.VMEM((_BATCH + 256,), jnp.int32),
        pltpu.VMEM((_BATCH + 256,), jnp.int32),
        pltpu.VMEM((128,), jnp.int32),
        pltpu.VMEM((128,), jnp.int32),
        pltpu.VMEM((128,), jnp.int32),
        pltpu.VMEM((128,), jnp.int32),
        pltpu.VMEM((128,), jnp.int32),
        pltpu.VMEM((128,), jnp.int32),
        pltpu.VMEM((_HID, _CW), jnp.float32),
        pltpu.VMEM((_HID, _CW), jnp.float32),
        pltpu.VMEM((72, 128), jnp.float32),
        pltpu.VMEM((128, 128), jnp.float32),
        pltpu.SemaphoreType.DMA,
        pltpu.SemaphoreType.DMA,
        pltpu.SemaphoreType.DMA,
    ],
    compiler_params=pltpu.CompilerParams(needs_layout_passes=False),
)
def _embed(lab_hbm, drop_hbm, table_t, tail_hbm, out_hbm,
           lab_v, drop_v, slab_v, spos_v, cnts_v, cnts_b, cnts_c, cnts_d,
           offs_v, posb_v, buf0, buf1, tail_v, extbuf, sem0, sem1, semf):
    wid = lax.axis_index("s") * _NC + lax.axis_index("c")
    is31 = wid == _NW - 1
    lo = wid * _RANGE
    hi = jnp.where(is31, _NUM_CLASSES + 1, lo + _RANGE)
    nch = jnp.where(is31, 62, 61)
    ones = jnp.ones((_L,), jnp.int32)
    negones = _full(-1)

    def issue(c, buf, sem):
        col = pl.multiple_of(lo + c * _CW, _CW)
        return pltpu.async_copy(table_t.at[:, pl.ds(col, _CW)], buf, sem)

    issue(0, buf0, sem0)
    issue(1, buf1, sem1)

    pltpu.sync_copy(lab_hbm, lab_v)
    pltpu.sync_copy(drop_hbm, drop_v)
    pltpu.sync_copy(tail_hbm, tail_v)

    for k in range(8):
        cnts_v[pl.ds(k * _L, _L)] = jnp.zeros((_L,), jnp.int32)
    for ref in (cnts_b, cnts_c, cnts_d):
        for k in range(4):
            ref[pl.ds(k * _L, _L)] = jnp.zeros((_L,), jnp.int32)

    def in_range_q(idx):
        inr = (idx >= lo) & (idx < hi)
        q = jnp.minimum(lax.shift_right_logical(idx - lo, 9), 62)
        q = jnp.where(inr, q, 0)
        return inr, q

    stripes = (cnts_v, cnts_b, cnts_c, cnts_d)

    def hist_body(k4, _):
        for u in range(8):
            k = k4 * 8 + u
            sl = pl.ds(k * _L, _L)
            idx = jnp.where(drop_v[sl] != 0, _NUM_CLASSES, lab_v[sl])
            lab_v[sl] = idx
            inr, q = in_range_q(idx)
            plsc.addupdate_scatter(stripes[u % 4], [q], ones, mask=inr)
        return 0
    lax.fori_loop(0, _BATCH // _L // 8, hist_body, 0)

    for k in range(4):
        sl = pl.ds(k * _L, _L)
        cnts_v[sl] = cnts_v[sl] + cnts_b[sl] + cnts_c[sl] + cnts_d[sl]

    carry = jnp.int32(0)
    for k in range(8):
        sl = pl.ds(k * _L, _L)
        c = cnts_v[sl]
        offs_v[sl] = plsc.cumsum(c) - c + carry
        carry = carry + jnp.sum(c)

    base_rank = jnp.min(plsc.scan_count(jnp.zeros((_L,), jnp.int32))[0])

    def place_body(k4, _):
        for u in range(8):
            k = k4 * 8 + u
            idx = lab_v[pl.ds(k * _L, _L)]
            inr, q = in_range_q(idx)
            gpos = k * _L + _iota()
            base = plsc.load_gather(offs_v, [q])
            rank = plsc.scan_count(q, mask=inr)[0] - base_rank
            slot = base + rank
            plsc.store_scatter(slab_v, [slot], idx, mask=inr)
            plsc.store_scatter(spos_v, [slot], gpos, mask=inr)
            plsc.addupdate_scatter(offs_v, [q], ones, mask=inr)
        return 0
    lax.fori_loop(0, _BATCH // _L // 8, place_body, 0)

    for k in range(8):
        posb_v[pl.ds(k * _L, _L)] = negones

    def flush():
        pltpu.async_copy(
            extbuf, out_hbm.at[plsc.Indices(posb_v, ignored_value=-1)], semf
        ).wait()
        for k in range(8):
            posb_v[pl.ds(k * _L, _L)] = negones

    def cnt_at(c):
        acc = jnp.int32(0)
        for k in range(4):
            acc = acc + jnp.sum(jnp.where(k * _L + _iota() == c,
                                          cnts_v[pl.ds(k * _L, _L)], 0))
        return acc

    def do_segment(seg_lo, seg_hi, j, src, base_col, transposed):
        a0 = (seg_lo // 8) * 8

        def chunk_body(m, j):
            base_i = a0 + _L * m
            lpos = base_i + _iota()
            msk = (lpos >= seg_lo) & (lpos < seg_hi)
            sl = pl.ds(pl.multiple_of(base_i, 8), _L)
            labs = slab_v[sl]
            poss = spos_v[sl]
            r = labs - base_col
            mi = msk.astype(jnp.int32)
            slotv = j + plsc.cumsum(mi) - 1
            for h in range(_HID):
                hv = _full(h)
                if transposed:
                    g = plsc.load_gather(src, [hv, r], mask=msk)
                else:
                    g = plsc.load_gather(src, [r, hv], mask=msk)
                plsc.store_scatter(extbuf, [slotv, hv], g, mask=msk)
            plsc.store_scatter(posb_v, [slotv], poss, mask=msk)
            j = j + jnp.sum(mi)
            pl.when(j > 128 - _L)(flush)
            return jnp.where(j > 128 - _L, 0, j)

        nchk = jnp.where(seg_hi > seg_lo, (seg_hi - a0 + _L - 1) // _L, 0)
        return lax.fori_loop(0, nchk, chunk_body, j)

    def pair_body(t, carry):
        j, s_lo = carry
        c0 = 2 * t
        pltpu.make_async_copy(table_t.at[:, pl.ds(0, _CW)], buf0, sem0).wait()
        s_hi = s_lo + cnt_at(c0)
        j = do_segment(s_lo, s_hi, j, buf0, lo + c0 * _CW, True)
        s_lo = s_hi

        @pl.when(c0 + 2 < nch)
        def _():
            issue(c0 + 2, buf0, sem0)

        has_odd = c0 + 1 < nch

        @pl.when(has_odd)
        def _():
            pltpu.make_async_copy(
                table_t.at[:, pl.ds(0, _CW)], buf1, sem1).wait()

        s_hi = s_lo + jnp.where(has_odd, cnt_at(c0 + 1), 0)
        j = do_segment(s_lo, s_hi, j, buf1, lo + (c0 + 1) * _CW, True)
        s_lo = s_hi

        @pl.when(c0 + 3 < nch)
        def _():
            issue(c0 + 3, buf1, sem1)

        return (j, s_lo)

    j, s_lo = lax.fori_loop(0, (nch + 1) // 2, pair_body,
                            (jnp.int32(0), jnp.int32(0)))

    j = do_segment(s_lo, s_lo + cnt_at(62), j, tail_v, _TAIL_LO, False)

    pl.when(j > 0)(flush)


def kernel(labels, train, force_drop_ids, table):
    lab = labels.astype(jnp.int32)
    drop = force_drop_ids.astype(jnp.int32)
    table_t = jnp.swapaxes(table, 0, 1)
    tail = lax.pad(
        lax.slice(table, (_TAIL_LO, 0), (_NUM_CLASSES + 1, _HID)),
        jnp.float32(0.0), ((0, 7, 0), (0, 96, 0)))
    out_wide = _embed(lab, drop, table_t, tail)
    return lax.slice(out_wide, (0, 0), (_BATCH, _HID))
